# baseline (speedup 1.0000x reference)
import functools

import jax
import jax.numpy as jnp
from jax import lax
from jax.experimental import pallas as pl
from jax.experimental.pallas import tpu as pltpu
from jax.experimental.pallas import tpu_sc as plsc

B = 1024
L = 200
D = 64
NC = 2
NS = 16
NW = NC * NS
NBB = 8
BBLK = B // NBB
NLG = NW // NBB
LBLK = L // NLG
LANES = 16


def _positional_encoding(max_seq_len, d_model):
    even_i = jnp.arange(0, d_model, 2, dtype=jnp.float32)
    denominator = jnp.power(10000.0, even_i / d_model)
    pos = jnp.arange(max_seq_len, dtype=jnp.float32).reshape(max_seq_len, 1)
    even_pe = jnp.sin(pos / denominator)
    odd_pe = jnp.cos(pos / denominator)
    stacked = jnp.stack([even_pe, odd_pe], axis=2)
    return stacked.reshape(max_seq_len, d_model)


def _make_sc_call():
    mesh = plsc.VectorSubcoreMesh(core_axis_name="c", subcore_axis_name="s")

    @functools.partial(
        pl.kernel,
        mesh=mesh,
        out_type=jax.ShapeDtypeStruct((L, D // 8, NBB, 8, BBLK), jnp.float32),
        compiler_params=pltpu.CompilerParams(
            use_tc_tiling_on_sc=False, needs_layout_passes=False),
        scratch_types=[
            pltpu.VMEM((LBLK, BBLK), jnp.int32),
            pltpu.VMEM((LBLK, D), jnp.float32),
            pltpu.VMEM((BBLK, D), jnp.float32),
            pltpu.VMEM((BBLK, D), jnp.float32),
            pltpu.VMEM((D // 8, 8, BBLK), jnp.float32),
            pltpu.VMEM((D // 8, 8, BBLK), jnp.float32),
            pltpu.SemaphoreType.DMA,
            pltpu.SemaphoreType.DMA,
            pltpu.SemaphoreType.DMA,
            pltpu.SemaphoreType.DMA,
        ],
    )
    def sc_embed(table_h, idxt_h, pe_h, out_h,
                 idx_v, pe_v, g0, g1, t0, t1, gs0, gs1, ss0, ss1):
        wid = lax.axis_index("s") * NC + lax.axis_index("c")
        bt = wid % NBB
        l0 = (wid // NBB) * LBLK
        pltpu.sync_copy(
            idxt_h.at[pl.ds(l0, LBLK), pl.ds(bt * BBLK, BBLK)], idx_v)
        pltpu.sync_copy(pe_h.at[pl.ds(l0, LBLK)], pe_v)

        gbufs, tbufs = (g0, g1), (t0, t1)
        gsems, ssems = (gs0, gs1), (ss0, ss1)

        iota = lax.iota(jnp.int32, LANES)
        ctv = [(16 * j + iota) // 8 for j in range(D // LANES)]
        csv = [(16 * j + iota) % 8 for j in range(D // LANES)]

        def fire_gather(li, gbuf, gsem):
            pltpu.async_copy(table_h.at[idx_v.at[li]], gbuf, gsem)

        def wait_gather(li, gbuf, gsem):
            pltpu.make_async_copy(table_h.at[idx_v.at[li]], gbuf, gsem).wait()

        def out_slice(li):
            return out_h.at[l0 + li, :, bt]

        def fire_scatter(li, tbuf, ssem):
            pltpu.async_copy(tbuf, out_slice(li), ssem)

        def wait_scatter(li, tbuf, ssem):
            pltpu.make_async_copy(tbuf, out_slice(li), ssem).wait()

        fire_gather(0, g0, gs0)
        fire_gather(1, g1, gs1)

        @pl.loop(0, LBLK, step=2)
        def per_pair(li0):
            for b in range(2):
                li = li0 + b
                gbuf, tbuf = gbufs[b], tbufs[b]
                gsem, ssem = gsems[b], ssems[b]
                wait_gather(li, gbuf, gsem)

                @pl.when(li >= 2)
                def _():
                    wait_scatter(li, tbuf, ssem)

                pe_rows = [pe_v[li, pl.ds(j * LANES, LANES)]
                           for j in range(D // LANES)]

                @plsc.parallel_loop(0, BBLK, unroll=2)
                def per_token(i):
                    blv = jnp.full((LANES,), i, jnp.int32)
                    for j in range(D // LANES):
                        val = gbuf[i, pl.ds(j * LANES, LANES)] + pe_rows[j]
                        plsc.store_scatter(tbuf, [ctv[j], csv[j], blv], val)

                @pl.when(li + 2 < LBLK)
                def _():
                    fire_gather(li + 2, gbuf, gsem)

                fire_scatter(li, tbuf, ssem)

        wait_scatter(LBLK - 2, t0, ss0)
        wait_scatter(LBLK - 1, t1, ss1)

    return sc_embed


_sc_embed = _make_sc_call()


def kernel(x, table):
    pe = _positional_encoding(L, D)
    idxt = x.T
    out5 = _sc_embed(table, idxt, pe)
    return out5.transpose(2, 4, 0, 1, 3).reshape(B, L, D)


# speedup vs baseline: 2.0727x; 2.0727x over previous
import functools

import jax
import jax.numpy as jnp
from jax import lax
from jax.experimental import pallas as pl
from jax.experimental.pallas import tpu as pltpu
from jax.experimental.pallas import tpu_sc as plsc

B = 1024
L = 200
D = 64
NC = 2
NS = 16
NW = NC * NS
NBB = 8
BBLK = B // NBB
NLG = NW // NBB
LBLK = L // NLG
LANES = 16


def _positional_encoding(max_seq_len, d_model):
    even_i = jnp.arange(0, d_model, 2, dtype=jnp.float32)
    denominator = jnp.power(10000.0, even_i / d_model)
    pos = jnp.arange(max_seq_len, dtype=jnp.float32).reshape(max_seq_len, 1)
    even_pe = jnp.sin(pos / denominator)
    odd_pe = jnp.cos(pos / denominator)
    stacked = jnp.stack([even_pe, odd_pe], axis=2)
    return stacked.reshape(max_seq_len, d_model)


def _make_sc_call():
    mesh = plsc.VectorSubcoreMesh(core_axis_name="c", subcore_axis_name="s")

    @functools.partial(
        pl.kernel,
        mesh=mesh,
        out_type=jax.ShapeDtypeStruct((L, D // 8, NBB, 8, BBLK), jnp.float32),
        compiler_params=pltpu.CompilerParams(
            use_tc_tiling_on_sc=False, needs_layout_passes=False),
        scratch_types=[
            pltpu.VMEM((LBLK, BBLK), jnp.int32),
            pltpu.VMEM((LBLK, D), jnp.float32),
            pltpu.VMEM((BBLK, D), jnp.float32),
            pltpu.VMEM((BBLK, D), jnp.float32),
            pltpu.VMEM((D // 8, 8, BBLK + 1), jnp.float32),
            pltpu.VMEM((D // 8, 8, BBLK + 1), jnp.float32),
            pltpu.SemaphoreType.DMA,
            pltpu.SemaphoreType.DMA,
            pltpu.SemaphoreType.DMA,
            pltpu.SemaphoreType.DMA,
        ],
    )
    def sc_embed(table_h, idxt_h, pe_h, out_h,
                 idx_v, pe_v, g0, g1, t0, t1, gs0, gs1, ss0, ss1):
        wid = lax.axis_index("s") * NC + lax.axis_index("c")
        bt = wid % NBB
        l0 = (wid // NBB) * LBLK
        pltpu.sync_copy(
            idxt_h.at[pl.ds(l0, LBLK), pl.ds(bt * BBLK, BBLK)], idx_v)
        pltpu.sync_copy(pe_h.at[pl.ds(l0, LBLK)], pe_v)

        gbufs, tbufs = (g0, g1), (t0, t1)
        gsems, ssems = (gs0, gs1), (ss0, ss1)

        iota = lax.iota(jnp.int32, LANES)
        ctv = [(16 * j + iota) // 8 for j in range(D // LANES)]
        csv = [(16 * j + iota) % 8 for j in range(D // LANES)]

        def fire_gather(li, gbuf, gsem):
            pltpu.async_copy(table_h.at[idx_v.at[li]], gbuf, gsem)

        def wait_gather(li, gbuf, gsem):
            pltpu.make_async_copy(table_h.at[idx_v.at[li]], gbuf, gsem).wait()

        def out_slice(li):
            return out_h.at[l0 + li, :, bt]

        def fire_scatter(li, tbuf, ssem):
            pltpu.async_copy(
                tbuf.at[:, :, pl.ds(0, BBLK)], out_slice(li), ssem)

        def wait_scatter(li, tbuf, ssem):
            pltpu.make_async_copy(
                tbuf.at[:, :, pl.ds(0, BBLK)], out_slice(li), ssem).wait()

        fire_gather(0, g0, gs0)
        fire_gather(1, g1, gs1)

        @pl.loop(0, LBLK, step=2)
        def per_pair(li0):
            for b in range(2):
                li = li0 + b
                gbuf, tbuf = gbufs[b], tbufs[b]
                gsem, ssem = gsems[b], ssems[b]
                wait_gather(li, gbuf, gsem)

                @pl.when(li >= 2)
                def _():
                    wait_scatter(li, tbuf, ssem)

                pe_rows = [pe_v[li, pl.ds(j * LANES, LANES)]
                           for j in range(D // LANES)]

                @plsc.parallel_loop(0, BBLK, unroll=2)
                def per_token(i):
                    blv = jnp.full((LANES,), i, jnp.int32)
                    for j in range(D // LANES):
                        val = gbuf[i, pl.ds(j * LANES, LANES)] + pe_rows[j]
                        plsc.store_scatter(tbuf, [ctv[j], csv[j], blv], val)

                @pl.when(li + 2 < LBLK)
                def _():
                    fire_gather(li + 2, gbuf, gsem)

                fire_scatter(li, tbuf, ssem)

        wait_scatter(LBLK - 2, t0, ss0)
        wait_scatter(LBLK - 1, t1, ss1)

    return sc_embed


_sc_embed = _make_sc_call()


def kernel(x, table):
    pe = _positional_encoding(L, D)
    idxt = x.T
    out5 = _sc_embed(table, idxt, pe)
    return out5.transpose(2, 4, 0, 1, 3).reshape(B, L, D)


# speedup vs baseline: 2.1772x; 1.0504x over previous
import functools

import jax
import jax.numpy as jnp
from jax import lax
from jax.experimental import pallas as pl
from jax.experimental.pallas import tpu as pltpu
from jax.experimental.pallas import tpu_sc as plsc

B = 1024
L = 200
D = 64
NC = 2
NS = 16
NW = NC * NS
NBB = 8
BBLK = B // NBB
NLG = NW // NBB
LBLK = L // NLG
LANES = 16
NBUF = 5


def _positional_encoding(max_seq_len, d_model):
    even_i = jnp.arange(0, d_model, 2, dtype=jnp.float32)
    denominator = jnp.power(10000.0, even_i / d_model)
    pos = jnp.arange(max_seq_len, dtype=jnp.float32).reshape(max_seq_len, 1)
    even_pe = jnp.sin(pos / denominator)
    odd_pe = jnp.cos(pos / denominator)
    stacked = jnp.stack([even_pe, odd_pe], axis=2)
    return stacked.reshape(max_seq_len, d_model)


def _make_sc_call():
    mesh = plsc.VectorSubcoreMesh(core_axis_name="c", subcore_axis_name="s")

    scratch = [pltpu.VMEM((LBLK, BBLK), jnp.int32),
               pltpu.VMEM((LBLK, D), jnp.float32)]
    scratch += [pltpu.VMEM((BBLK, D), jnp.float32) for _ in range(NBUF)]
    scratch += [pltpu.VMEM((D // 8, 8, BBLK + 1), jnp.float32)
                for _ in range(NBUF)]
    scratch += [pltpu.SemaphoreType.DMA for _ in range(2 * NBUF)]

    @functools.partial(
        pl.kernel,
        mesh=mesh,
        out_type=jax.ShapeDtypeStruct((L, D // 8, NBB, 8, BBLK), jnp.float32),
        compiler_params=pltpu.CompilerParams(
            use_tc_tiling_on_sc=False, needs_layout_passes=False),
        scratch_types=scratch,
    )
    def sc_embed(table_h, idxt_h, pe_h, out_h, idx_v, pe_v, *bufs):
        gbufs = bufs[:NBUF]
        tbufs = bufs[NBUF:2 * NBUF]
        gsems = bufs[2 * NBUF:3 * NBUF]
        ssems = bufs[3 * NBUF:4 * NBUF]

        wid = lax.axis_index("s") * NC + lax.axis_index("c")
        bt = wid % NBB
        l0 = (wid // NBB) * LBLK
        pltpu.sync_copy(
            idxt_h.at[pl.ds(l0, LBLK), pl.ds(bt * BBLK, BBLK)], idx_v)
        pltpu.sync_copy(pe_h.at[pl.ds(l0, LBLK)], pe_v)

        iota = lax.iota(jnp.int32, LANES)
        ctv = [(16 * j + iota) // 8 for j in range(D // LANES)]
        csv = [(16 * j + iota) % 8 for j in range(D // LANES)]

        def fire_gather(li, b):
            pltpu.async_copy(table_h.at[idx_v.at[li]], gbufs[b], gsems[b])

        def wait_gather(li, b):
            pltpu.make_async_copy(
                table_h.at[idx_v.at[li]], gbufs[b], gsems[b]).wait()

        def out_slice(li):
            return out_h.at[l0 + li, :, bt]

        def fire_scatter(li, b):
            pltpu.async_copy(
                tbufs[b].at[:, :, pl.ds(0, BBLK)], out_slice(li), ssems[b])

        def wait_scatter(li, b):
            pltpu.make_async_copy(
                tbufs[b].at[:, :, pl.ds(0, BBLK)], out_slice(li),
                ssems[b]).wait()

        for b in range(NBUF):
            fire_gather(b, b)

        @pl.loop(0, LBLK, step=NBUF)
        def per_group(li0):
            for b in range(NBUF):
                li = li0 + b
                wait_gather(li, b)

                @pl.when(li >= NBUF)
                def _():
                    wait_scatter(li, b)

                pe_rows = [pe_v[li, pl.ds(j * LANES, LANES)]
                           for j in range(D // LANES)]

                @plsc.parallel_loop(0, BBLK, unroll=4)
                def per_token(i):
                    blv = jnp.full((LANES,), i, jnp.int32)
                    for j in range(D // LANES):
                        val = gbufs[b][i, pl.ds(j * LANES, LANES)] + pe_rows[j]
                        plsc.store_scatter(
                            tbufs[b], [ctv[j], csv[j], blv], val)

                @pl.when(li + NBUF < LBLK)
                def _():
                    fire_gather(li + NBUF, b)

                fire_scatter(li, b)

        for b in range(NBUF):
            wait_scatter(LBLK - NBUF + b, b)

    return sc_embed


_sc_embed = _make_sc_call()


def kernel(x, table):
    pe = _positional_encoding(L, D)
    idxt = x.T
    out5 = _sc_embed(table, idxt, pe)
    return out5.transpose(2, 4, 0, 1, 3).reshape(B, L, D)


# speedup vs baseline: 2.2207x; 1.0200x over previous
import functools

import jax
import jax.numpy as jnp
from jax import lax
from jax.experimental import pallas as pl
from jax.experimental.pallas import tpu as pltpu
from jax.experimental.pallas import tpu_sc as plsc

B = 1024
L = 200
D = 64
NC = 2
NS = 16
NW = NC * NS
NBB = 8
BBLK = B // NBB
NLG = NW // NBB
LBLK = L // NLG
LANES = 16
NBUF = 2


def _positional_encoding(max_seq_len, d_model):
    even_i = jnp.arange(0, d_model, 2, dtype=jnp.float32)
    denominator = jnp.power(10000.0, even_i / d_model)
    pos = jnp.arange(max_seq_len, dtype=jnp.float32).reshape(max_seq_len, 1)
    even_pe = jnp.sin(pos / denominator)
    odd_pe = jnp.cos(pos / denominator)
    stacked = jnp.stack([even_pe, odd_pe], axis=2)
    return stacked.reshape(max_seq_len, d_model)


def _make_sc_call():
    mesh = plsc.VectorSubcoreMesh(core_axis_name="c", subcore_axis_name="s")

    scratch = [pltpu.VMEM((LBLK, BBLK), jnp.int32),
               pltpu.VMEM((LBLK, D), jnp.float32)]
    scratch += [pltpu.VMEM((BBLK, 2 * D), jnp.float32) for _ in range(NBUF)]
    scratch += [pltpu.VMEM((D // 8, 8, BBLK + 1), jnp.float32)
                for _ in range(NBUF)]
    scratch += [pltpu.SemaphoreType.DMA for _ in range(2 * NBUF)]

    @functools.partial(
        pl.kernel,
        mesh=mesh,
        out_type=jax.ShapeDtypeStruct((L, D // 8, NBB, 8, BBLK), jnp.float32),
        compiler_params=pltpu.CompilerParams(
            use_tc_tiling_on_sc=False, needs_layout_passes=False),
        scratch_types=scratch,
    )
    def sc_embed(table_h, idxt_h, pe_h, out_h, idx_v, pe_v, *bufs):
        gbufs = bufs[:NBUF]
        tbufs = bufs[NBUF:2 * NBUF]
        gsems = bufs[2 * NBUF:3 * NBUF]
        ssems = bufs[3 * NBUF:4 * NBUF]

        wid = lax.axis_index("s") * NC + lax.axis_index("c")
        bt = wid % NBB
        l0 = (wid // NBB) * LBLK
        pltpu.sync_copy(
            idxt_h.at[pl.ds(l0, LBLK), pl.ds(bt * BBLK, BBLK)], idx_v)
        pltpu.sync_copy(pe_h.at[pl.ds(l0, LBLK)], pe_v)

        iota = lax.iota(jnp.int32, LANES)
        ctv = [(16 * j + iota) // 8 for j in range(D // LANES)]
        csv = [(16 * j + iota) % 8 for j in range(D // LANES)]

        def fire_gather(li, b):
            pltpu.async_copy(table_h.at[idx_v.at[li]], gbufs[b], gsems[b])

        def wait_gather(li, b):
            pltpu.make_async_copy(
                table_h.at[idx_v.at[li]], gbufs[b], gsems[b]).wait()

        def out_slice(li):
            return out_h.at[l0 + li, :, bt]

        def fire_scatter(li, b):
            pltpu.async_copy(
                tbufs[b].at[:, :, pl.ds(0, BBLK)], out_slice(li), ssems[b])

        def wait_scatter(li, b):
            pltpu.make_async_copy(
                tbufs[b].at[:, :, pl.ds(0, BBLK)], out_slice(li),
                ssems[b]).wait()

        for b in range(NBUF):
            fire_gather(b, b)

        @pl.loop(0, LBLK, step=NBUF)
        def per_group(li0):
            for b in range(NBUF):
                li = li0 + b
                wait_gather(li, b)

                @pl.when(li >= NBUF)
                def _():
                    wait_scatter(li, b)

                pe_rows = [pe_v[li, pl.ds(j * LANES, LANES)]
                           for j in range(D // LANES)]

                @plsc.parallel_loop(0, BBLK, unroll=4)
                def per_token(i):
                    blv = jnp.full((LANES,), i, jnp.int32)
                    for j in range(D // LANES):
                        val = gbufs[b][i, pl.ds(j * LANES, LANES)] + pe_rows[j]
                        plsc.store_scatter(
                            tbufs[b], [ctv[j], csv[j], blv], val)

                @pl.when(li + NBUF < LBLK)
                def _():
                    fire_gather(li + NBUF, b)

                fire_scatter(li, b)

        for b in range(NBUF):
            wait_scatter(LBLK - NBUF + b, b)

    return sc_embed


_sc_embed = _make_sc_call()

VB = 8192


def _tc_transpose_body(tT_ref, out_ref):
    t = tT_ref[...].T
    out_ref[:, 0:D] = t
    out_ref[:, D:2 * D] = t


_tc_transpose = pl.pallas_call(
    _tc_transpose_body,
    grid=((100000 + VB - 1) // VB,),
    in_specs=[pl.BlockSpec((D, VB), lambda i: (0, i))],
    out_specs=pl.BlockSpec((VB, 2 * D), lambda i: (i, 0)),
    out_shape=jax.ShapeDtypeStruct((100000, 2 * D), jnp.float32),
)


def kernel(x, table):
    pe = _positional_encoding(L, D)
    idxt = x.T
    t128 = _tc_transpose(table.T)
    out5 = _sc_embed(t128, idxt, pe)
    return out5.transpose(2, 4, 0, 1, 3).reshape(B, L, D)


# speedup vs baseline: 2.9580x; 1.3320x over previous
import functools

import jax
import jax.numpy as jnp
from jax import lax
from jax.experimental import pallas as pl
from jax.experimental.pallas import tpu as pltpu
from jax.experimental.pallas import tpu_sc as plsc

B = 1024
L = 200
D = 64
NC = 2
NS = 16
NW = NC * NS
NBB = 8
BBLK = B // NBB
NLG = NW // NBB
LBLK = L // NLG
LANES = 16
NBUF = 5


def _positional_encoding(max_seq_len, d_model):
    even_i = jnp.arange(0, d_model, 2, dtype=jnp.float32)
    denominator = jnp.power(10000.0, even_i / d_model)
    pos = jnp.arange(max_seq_len, dtype=jnp.float32).reshape(max_seq_len, 1)
    even_pe = jnp.sin(pos / denominator)
    odd_pe = jnp.cos(pos / denominator)
    stacked = jnp.stack([even_pe, odd_pe], axis=2)
    return stacked.reshape(max_seq_len, d_model)


def _make_sc_call():
    mesh = plsc.VectorSubcoreMesh(core_axis_name="c", subcore_axis_name="s")

    scratch = [pltpu.VMEM((LBLK, BBLK), jnp.int32),
               pltpu.VMEM((LBLK, D), jnp.float32)]
    scratch += [pltpu.VMEM((BBLK, D), jnp.float32) for _ in range(NBUF)]
    scratch += [pltpu.VMEM((D // 8, 8, BBLK + 1), jnp.float32)
                for _ in range(NBUF)]
    scratch += [pltpu.SemaphoreType.DMA for _ in range(2 * NBUF)]

    @functools.partial(
        pl.kernel,
        mesh=mesh,
        out_type=jax.ShapeDtypeStruct((L, D // 8, NBB, 8, BBLK), jnp.float32),
        compiler_params=pltpu.CompilerParams(
            use_tc_tiling_on_sc=False, needs_layout_passes=False),
        scratch_types=scratch,
    )
    def sc_embed(table_h, idxt_h, pe_h, out_h, idx_v, pe_v, *bufs):
        gbufs = bufs[:NBUF]
        tbufs = bufs[NBUF:2 * NBUF]
        gsems = bufs[2 * NBUF:3 * NBUF]
        ssems = bufs[3 * NBUF:4 * NBUF]

        wid = lax.axis_index("s") * NC + lax.axis_index("c")
        bt = wid % NBB
        l0 = (wid // NBB) * LBLK
        pltpu.sync_copy(
            idxt_h.at[pl.ds(l0, LBLK), pl.ds(bt * BBLK, BBLK)], idx_v)
        pltpu.sync_copy(pe_h.at[pl.ds(l0, LBLK)], pe_v)

        iota = lax.iota(jnp.int32, LANES)
        ctv = [(16 * j + iota) // 8 for j in range(D // LANES)]
        csv = [(16 * j + iota) % 8 for j in range(D // LANES)]

        def fire_gather(li, b):
            pltpu.async_copy(table_h.at[idx_v.at[li]], gbufs[b], gsems[b])

        def wait_gather(li, b):
            pltpu.make_async_copy(
                table_h.at[idx_v.at[li]], gbufs[b], gsems[b]).wait()

        def out_slice(li):
            return out_h.at[l0 + li, :, bt]

        def fire_scatter(li, b):
            pltpu.async_copy(
                tbufs[b].at[:, :, pl.ds(0, BBLK)], out_slice(li), ssems[b])

        def wait_scatter(li, b):
            pltpu.make_async_copy(
                tbufs[b].at[:, :, pl.ds(0, BBLK)], out_slice(li),
                ssems[b]).wait()

        for b in range(NBUF):
            fire_gather(b, b)

        @pl.loop(0, LBLK, step=NBUF)
        def per_group(li0):
            for b in range(NBUF):
                li = li0 + b
                wait_gather(li, b)

                @pl.when(li >= NBUF)
                def _():
                    wait_scatter(li, b)

                pe_rows = [pe_v[li, pl.ds(j * LANES, LANES)]
                           for j in range(D // LANES)]

                @plsc.parallel_loop(0, BBLK, unroll=4)
                def per_token(i):
                    blv = jnp.full((LANES,), i, jnp.int32)
                    for j in range(D // LANES):
                        val = gbufs[b][i, pl.ds(j * LANES, LANES)] + pe_rows[j]
                        plsc.store_scatter(
                            tbufs[b], [ctv[j], csv[j], blv], val)

                @pl.when(li + NBUF < LBLK)
                def _():
                    fire_gather(li + NBUF, b)

                fire_scatter(li, b)

        for b in range(NBUF):
            wait_scatter(LBLK - NBUF + b, b)

    return sc_embed


_sc_embed = _make_sc_call()

VB = 8192


def _tc_transpose_body(tT_ref, out_ref):
    out_ref[:, 0:D] = tT_ref[...].T


_tc_transpose = pl.pallas_call(
    _tc_transpose_body,
    grid=((100000 + VB - 1) // VB,),
    in_specs=[pl.BlockSpec((D, VB), lambda i: (0, i))],
    out_specs=pl.BlockSpec((VB, 2 * D), lambda i: (i, 0)),
    out_shape=jax.ShapeDtypeStruct((100000, 2 * D), jnp.float32),
)


def kernel(x, table):
    pe = _positional_encoding(L, D)
    idxt = (x * 2).T
    t128 = _tc_transpose(table.T)
    out5 = _sc_embed(t128.reshape(2 * 100000, D), idxt, pe)
    return out5.transpose(2, 4, 0, 1, 3).reshape(B, L, D)
